# Initial kernel scaffold; baseline (speedup 1.0000x reference)
import jax
import jax.numpy as jnp
from jax.experimental import pallas as pl


def kernel(features, labels, prototypes):
    raise NotImplementedError("write your pallas kernel here")


# speedup vs baseline: 1.4060x; 1.4060x over previous
import functools

import jax
import jax.numpy as jnp
from jax import lax
from jax.experimental import pallas as pl
from jax.experimental.pallas import tpu as pltpu
from jax.experimental.pallas import tpu_sc as plsc

_B = 16384
_D = 128
_C = 1000
_CP = 1024
_MOM = 0.99
_EPS = 1e-12

_NC = 2
_NS = 16
_NW = _NC * _NS
_BPW = _B // _NW
_RPT = _CP // _NS


def _norm_body(x_ref, o_ref):
    x = x_ref[...]
    n = jnp.sqrt(jnp.sum(x * x, axis=1, keepdims=True))
    o_ref[...] = x / jnp.maximum(n, _EPS)


_norm_feats = pl.pallas_call(
    _norm_body,
    grid=(16,),
    in_specs=[pl.BlockSpec((_B // 16, _D), lambda i: (i, 0))],
    out_specs=pl.BlockSpec((_B // 16, _D), lambda i: (i, 0)),
    out_shape=jax.ShapeDtypeStruct((_B, _D), jnp.float32),
)


_sc_mesh = plsc.VectorSubcoreMesh(core_axis_name="c", subcore_axis_name="s")


@functools.partial(
    pl.kernel,
    mesh=_sc_mesh,
    out_type=[
        jax.ShapeDtypeStruct((_NC * _CP, _D), jnp.float32),
        jax.ShapeDtypeStruct((_NC * _CP, 16), jnp.float32),
    ],
    scratch_types=[
        pltpu.VMEM((_BPW // 128, 128), jnp.int32),
        pltpu.VMEM((_BPW, _D), jnp.float32),
        pltpu.VMEM((128, 16), jnp.float32),
        pltpu.VMEM_SHARED((_CP, _D), jnp.float32),
        pltpu.VMEM_SHARED((_CP, 16), jnp.float32),
    ],
)
def _segsum(feats_hbm, labels_hbm, z128_hbm, z16_hbm, ones_hbm,
            sums_hbm, cnts_hbm, lbl_v, ft_v, ones_v, acc_s, cnt_s):
    c = lax.axis_index("c")
    s = lax.axis_index("s")
    wid = s * _NC + c
    r0 = s * _RPT
    pltpu.sync_copy(z128_hbm, acc_s.at[pl.ds(r0, _RPT)])
    pltpu.sync_copy(z16_hbm, cnt_s.at[pl.ds(r0, _RPT)])
    pltpu.sync_copy(feats_hbm.at[pl.ds(wid * _BPW, _BPW)], ft_v)
    pltpu.sync_copy(labels_hbm.at[pl.ds(wid * (_BPW // 128), _BPW // 128)], lbl_v)
    pltpu.sync_copy(ones_hbm, ones_v)
    plsc.subcore_barrier()
    for j in range(_BPW // 128):
        idx = lbl_v.at[j]
        pltpu.sync_copy(ft_v.at[pl.ds(j * 128, 128)], acc_s.at[idx], add=True)
        pltpu.sync_copy(ones_v, cnt_s.at[idx], add=True)
    plsc.subcore_barrier()
    o0 = c * _CP + r0
    pltpu.sync_copy(acc_s.at[pl.ds(r0, _RPT)], sums_hbm.at[pl.ds(o0, _RPT)])
    pltpu.sync_copy(cnt_s.at[pl.ds(r0, _RPT)], cnts_hbm.at[pl.ds(o0, _RPT)])


def _update_body(sums_ref, cnts_ref, pr_ref, p_ref, psq_ref):
    sums = sums_ref[0] + sums_ref[1]
    cnt = cnts_ref[0, :, 0:1] + cnts_ref[1, :, 0:1]
    pr = pr_ref[...]
    p0 = pr / jnp.maximum(jnp.sqrt(jnp.sum(pr * pr, axis=1, keepdims=True)), _EPS)
    mean = sums / jnp.maximum(cnt, 1.0)
    mean_n = mean / jnp.maximum(jnp.sqrt(jnp.sum(mean * mean, axis=1, keepdims=True)), _EPS)
    bl = _MOM * p0 + (1.0 - _MOM) * mean_n
    bl_n = bl / jnp.maximum(jnp.sqrt(jnp.sum(bl * bl, axis=1, keepdims=True)), _EPS)
    new = jnp.where(cnt > 0.0, bl_n, p0)
    p = new / jnp.maximum(jnp.sqrt(jnp.sum(new * new, axis=1, keepdims=True)), _EPS)
    p_ref[...] = p
    psq_ref[...] = lax.dot_general(
        jnp.ones((1, _D), jnp.float32), p * p,
        (((1,), (1,)), ((), ())), preferred_element_type=jnp.float32)


_update = pl.pallas_call(
    _update_body,
    in_specs=[
        pl.BlockSpec((_NC, _CP, _D), lambda: (0, 0, 0)),
        pl.BlockSpec((_NC, _CP, 16), lambda: (0, 0, 0)),
        pl.BlockSpec((_CP, _D), lambda: (0, 0)),
    ],
    out_specs=[
        pl.BlockSpec((_CP, _D), lambda: (0, 0)),
        pl.BlockSpec((1, _CP), lambda: (0, 0)),
    ],
    out_shape=[
        jax.ShapeDtypeStruct((_CP, _D), jnp.float32),
        jax.ShapeDtypeStruct((1, _CP), jnp.float32),
    ],
)


def _logits_body(f_ref, p_ref, psq_ref, o_ref):
    f = f_ref[...]
    fsq = jnp.sum(f * f, axis=1, keepdims=True)
    d = lax.dot_general(f, p_ref[...], (((1,), (1,)), ((), ())),
                        preferred_element_type=jnp.float32)
    sq = fsq + psq_ref[...] - 2.0 * d
    o_ref[...] = -jnp.sqrt(jnp.maximum(sq, 0.0))


_logits = pl.pallas_call(
    _logits_body,
    grid=(16,),
    in_specs=[
        pl.BlockSpec((_B // 16, _D), lambda i: (i, 0)),
        pl.BlockSpec((_C, _D), lambda i: (0, 0)),
        pl.BlockSpec((1, _C), lambda i: (0, 0)),
    ],
    out_specs=pl.BlockSpec((_B // 16, _C), lambda i: (i, 0)),
    out_shape=jax.ShapeDtypeStruct((_B, _C), jnp.float32),
)


def kernel(features, labels, prototypes):
    feats_n = _norm_feats(features)
    labels2d = labels.reshape(_B // 128, 128).astype(jnp.int32)
    z128 = jnp.zeros((_RPT, _D), jnp.float32)
    z16 = jnp.zeros((_RPT, 16), jnp.float32)
    ones = jnp.ones((128, 16), jnp.float32)
    sums, cnts = _segsum(feats_n, labels2d, z128, z16, ones)
    protos_pad = jnp.pad(prototypes, ((0, _CP - _C), (0, 0)))
    p, psq = _update(sums.reshape(_NC, _CP, _D), cnts.reshape(_NC, _CP, 16),
                     protos_pad)
    return _logits(feats_n, p[:_C], psq[:, :_C])


# speedup vs baseline: 1.4475x; 1.0295x over previous
import functools

import jax
import jax.numpy as jnp
from jax import lax
from jax.experimental import pallas as pl
from jax.experimental.pallas import tpu as pltpu
from jax.experimental.pallas import tpu_sc as plsc

_B = 16384
_D = 128
_C = 1000
_CP = 1024
_MOM = 0.99
_EPS = 1e-12

_NC = 2
_NS = 16
_NW = _NC * _NS
_BPW = _B // _NW
_RPT = _CP // _NS


def _norm_body(x_ref, o_ref):
    x = x_ref[...]
    n = jnp.sqrt(jnp.sum(x * x, axis=1, keepdims=True))
    o_ref[...] = x / jnp.maximum(n, _EPS)


_norm_feats = pl.pallas_call(
    _norm_body,
    grid=(16,),
    in_specs=[pl.BlockSpec((_B // 16, _D), lambda i: (i, 0))],
    out_specs=pl.BlockSpec((_B // 16, _D), lambda i: (i, 0)),
    out_shape=jax.ShapeDtypeStruct((_B, _D), jnp.float32),
)


_sc_mesh = plsc.VectorSubcoreMesh(core_axis_name="c", subcore_axis_name="s")


@functools.partial(
    pl.kernel,
    mesh=_sc_mesh,
    out_type=[
        jax.ShapeDtypeStruct((_NC * _CP, _D), jnp.float32),
        jax.ShapeDtypeStruct((_NC * _CP, 16), jnp.float32),
    ],
    scratch_types=[
        pltpu.VMEM((_BPW // 128, 128), jnp.int32),
        pltpu.VMEM((_BPW, _D), jnp.float32),
        pltpu.VMEM((128, 16), jnp.float32),
        pltpu.VMEM_SHARED((_CP, _D), jnp.float32),
        pltpu.VMEM_SHARED((_CP, 16), jnp.float32),
    ],
)
def _segsum(feats_hbm, labels_hbm, z128_hbm, z16_hbm, ones_hbm,
            sums_hbm, cnts_hbm, lbl_v, ft_v, ones_v, acc_s, cnt_s):
    c = lax.axis_index("c")
    s = lax.axis_index("s")
    wid = s * _NC + c
    r0 = s * _RPT
    pltpu.sync_copy(z128_hbm, acc_s.at[pl.ds(r0, _RPT)])
    pltpu.sync_copy(z16_hbm, cnt_s.at[pl.ds(r0, _RPT)])
    pltpu.sync_copy(feats_hbm.at[pl.ds(wid * _BPW, _BPW)], ft_v)
    pltpu.sync_copy(labels_hbm.at[pl.ds(wid * (_BPW // 128), _BPW // 128)], lbl_v)
    pltpu.sync_copy(ones_hbm, ones_v)
    plsc.subcore_barrier()
    for j in range(_BPW // 128):
        idx = lbl_v.at[j]
        pltpu.sync_copy(ft_v.at[pl.ds(j * 128, 128)], acc_s.at[idx], add=True)
        pltpu.sync_copy(ones_v, cnt_s.at[idx], add=True)
    plsc.subcore_barrier()
    o0 = c * _CP + r0
    pltpu.sync_copy(acc_s.at[pl.ds(r0, _RPT)], sums_hbm.at[pl.ds(o0, _RPT)])
    pltpu.sync_copy(cnt_s.at[pl.ds(r0, _RPT)], cnts_hbm.at[pl.ds(o0, _RPT)])


def _logits_body(f_ref, sums_ref, cnts_ref, pr_ref, o_ref, p_s, psq_s):
    @pl.when(pl.program_id(0) == 0)
    def _():
        sums = sums_ref[0] + sums_ref[1]
        cnt = cnts_ref[0, :, 0:1] + cnts_ref[1, :, 0:1]
        pr = pr_ref[...]
        p0 = pr / jnp.maximum(jnp.sqrt(jnp.sum(pr * pr, axis=1, keepdims=True)), _EPS)
        mean = sums / jnp.maximum(cnt, 1.0)
        mean_n = mean / jnp.maximum(jnp.sqrt(jnp.sum(mean * mean, axis=1, keepdims=True)), _EPS)
        bl = _MOM * p0 + (1.0 - _MOM) * mean_n
        bl_n = bl / jnp.maximum(jnp.sqrt(jnp.sum(bl * bl, axis=1, keepdims=True)), _EPS)
        new = jnp.where(cnt > 0.0, bl_n, p0)
        p = new / jnp.maximum(jnp.sqrt(jnp.sum(new * new, axis=1, keepdims=True)), _EPS)
        p_s[...] = p
        psq_s[...] = lax.dot_general(
            jnp.ones((1, _D), jnp.float32), p * p,
            (((1,), (1,)), ((), ())), preferred_element_type=jnp.float32)

    f = f_ref[...]
    fsq = jnp.sum(f * f, axis=1, keepdims=True)
    d = lax.dot_general(f, p_s[...], (((1,), (1,)), ((), ())),
                        preferred_element_type=jnp.float32)
    sq = fsq + psq_s[...] - 2.0 * d
    o_ref[...] = -jnp.sqrt(jnp.maximum(sq[:, :_C], 0.0))


_logits = pl.pallas_call(
    _logits_body,
    grid=(16,),
    in_specs=[
        pl.BlockSpec((_B // 16, _D), lambda i: (i, 0)),
        pl.BlockSpec((_NC, _CP, _D), lambda i: (0, 0, 0)),
        pl.BlockSpec((_NC, _CP, 16), lambda i: (0, 0, 0)),
        pl.BlockSpec((_CP, _D), lambda i: (0, 0)),
    ],
    out_specs=pl.BlockSpec((_B // 16, _C), lambda i: (i, 0)),
    out_shape=jax.ShapeDtypeStruct((_B, _C), jnp.float32),
    scratch_shapes=[
        pltpu.VMEM((_CP, _D), jnp.float32),
        pltpu.VMEM((1, _CP), jnp.float32),
    ],
)


def kernel(features, labels, prototypes):
    feats_n = _norm_feats(features)
    labels2d = labels.reshape(_B // 128, 128).astype(jnp.int32)
    z128 = jnp.zeros((_RPT, _D), jnp.float32)
    z16 = jnp.zeros((_RPT, 16), jnp.float32)
    ones = jnp.ones((128, 16), jnp.float32)
    sums, cnts = _segsum(feats_n, labels2d, z128, z16, ones)
    protos_pad = jnp.pad(prototypes, ((0, _CP - _C), (0, 0)))
    return _logits(feats_n, sums.reshape(_NC, _CP, _D),
                   cnts.reshape(_NC, _CP, 16), protos_pad)
